# baseline (speedup 1.0000x reference)
import jax
import jax.numpy as jnp
from jax import lax
from jax.experimental import pallas as pl
from jax.experimental.pallas import tpu as pltpu
from jax.experimental.pallas import tpu_sc as plsc

N = 10000
E = 320000
D = 128

NC = 2
NS = 16
NW = NC * NS
EPW = E // NW

DEG_C = 80
DEG_R = EPW // DEG_C

MSG_C = 128
MSG_R = 80
PAD = MSG_R * MSG_C - EPW
NA = N + 8
HH = MSG_R // 2
RPT = N // NS

_mesh = plsc.VectorSubcoreMesh(
    core_axis_name="c", subcore_axis_name="s", num_cores=NC, num_subcores=NS
)


def _deg_body(dst_hbm, degp_hbm, idx_v, deg_v, sem):
    cid = lax.axis_index("c")
    sid = lax.axis_index("s")
    w = cid * NS + sid
    pltpu.async_copy(dst_hbm.at[w], idx_v, sem).wait()

    zero16 = jnp.zeros((16,), jnp.float32)

    def zloop(i, carry):
        deg_v[pl.ds(i * 16, 16)] = zero16
        return carry

    lax.fori_loop(0, N // 16, zloop, 0)

    ones16 = jnp.ones((16,), jnp.float32)

    def eloop(r, carry):
        for g in range(DEG_C // 16):
            idx16 = idx_v[r, pl.ds(g * 16, 16)]
            plsc.addupdate_scatter(deg_v, [idx16], ones16)
        return carry

    lax.fori_loop(0, DEG_R, eloop, 0)
    pltpu.sync_copy(deg_v, degp_hbm.at[pl.ds(w * N, N)])


_deg_kernel = pl.kernel(
    _deg_body,
    out_type=jax.ShapeDtypeStruct((NW * N,), jnp.float32),
    mesh=_mesh,
    compiler_params=pltpu.CompilerParams(needs_layout_passes=False),
    scratch_types=[
        pltpu.VMEM((DEG_R, DEG_C), jnp.int32),
        pltpu.VMEM((N,), jnp.float32),
        pltpu.SemaphoreType.DMA,
    ],
)


ZCH = 16
RPT_A = 624
ZCH_N = RPT_A // ZCH


def _msg_body(hp_hbm, src_hbm, dst_hbm, out_hbm, sidx_v, didx_v, rows_v,
              agg_s, sem, ssem):
    cid = lax.axis_index("c")
    sid = lax.axis_index("s")
    w = cid * NS + sid
    pltpu.async_copy(src_hbm.at[w], sidx_v, sem)
    pltpu.async_copy(dst_hbm.at[w, pl.ds(0, HH)], didx_v, sem)

    zero16 = jnp.zeros((16,), jnp.float32)

    def zloop(i, carry):
        r = i // (D // 16)
        c = (i % (D // 16)) * 16
        rows_v[0, r, pl.ds(c, 16)] = zero16
        return carry

    lax.fori_loop(0, MSG_C * (D // 16), zloop, 0)
    base = sid * RPT_A
    nfull = RPT_A // MSG_C
    rem = RPT_A - nfull * MSG_C
    for k in range(nfull):
        pltpu.sync_copy(rows_v.at[0], agg_s.at[pl.ds(base + k * MSG_C, MSG_C)])
    pltpu.sync_copy(
        rows_v.at[0, pl.ds(0, rem)],
        agg_s.at[pl.ds(base + nfull * MSG_C, rem)],
    )

    @pl.when(sid == NS - 1)
    def _():
        pltpu.sync_copy(rows_v.at[0, pl.ds(0, ZCH)],
                        agg_s.at[pl.ds(N - ZCH, ZCH)])

    pltpu.make_async_copy(src_hbm.at[w], sidx_v, sem).wait()
    pltpu.make_async_copy(dst_hbm.at[w, pl.ds(0, HH)], didx_v, sem).wait()
    plsc.subcore_barrier()

    def pair_loop(first, kk, carry):
        for b in range(2):
            j = first + kk * 2 + b
            lr = j % HH
            pltpu.make_async_copy(
                rows_v.at[b], agg_s.at[didx_v.at[lr - 2]], ssem[b]
            ).wait()
            pltpu.async_copy(hp_hbm.at[sidx_v.at[j]], rows_v.at[b],
                             sem).wait()
            pltpu.async_copy(rows_v.at[b], agg_s.at[didx_v.at[lr]], ssem[b],
                             add=True)
        return carry

    for h in range(2):
        if h == 1:
            for b in range(2):
                pltpu.make_async_copy(
                    rows_v.at[b], agg_s.at[didx_v.at[HH - 2 + b]], ssem[b]
                ).wait()
            pltpu.sync_copy(dst_hbm.at[w, pl.ds(HH, HH)], didx_v)
        for b in range(2):
            j = h * HH + b
            pltpu.async_copy(hp_hbm.at[sidx_v.at[j]], rows_v.at[b],
                             sem).wait()
            pltpu.async_copy(rows_v.at[b], agg_s.at[didx_v.at[b]], ssem[b],
                             add=True)
        lax.fori_loop(
            0, (HH - 2) // 2,
            lambda kk, c, _f=h * HH + 2: pair_loop(_f, kk, c), 0
        )

    for b in range(2):
        pltpu.make_async_copy(
            rows_v.at[b], agg_s.at[didx_v.at[HH - 2 + b]], ssem[b]
        ).wait()
    plsc.subcore_barrier()
    pltpu.sync_copy(
        agg_s.at[pl.ds(base, RPT_A)],
        out_hbm.at[pl.ds(cid * N + base, RPT_A)],
    )

    @pl.when(sid == NS - 1)
    def _():
        pltpu.sync_copy(
            agg_s.at[pl.ds(N - ZCH, ZCH)],
            out_hbm.at[pl.ds(cid * N + N - ZCH, ZCH)],
        )


_msg_kernel = pl.kernel(
    _msg_body,
    out_type=jax.ShapeDtypeStruct((NC * N, D), jnp.float32),
    mesh=_mesh,
    scratch_types=[
        pltpu.VMEM((MSG_R, MSG_C), jnp.int32),
        pltpu.VMEM((HH, MSG_C), jnp.int32),
        pltpu.VMEM((2, MSG_C, D), jnp.float32),
        pltpu.VMEM_SHARED((NA, D), jnp.float32),
        pltpu.SemaphoreType.DMA,
        [pltpu.SemaphoreType.DMA] * 2,
    ],
)


RB = 1000
GRID = N // RB


def _dinv_body(degp_ref, dinv_ref):
    ones = jnp.ones((NW, D), jnp.float32)
    deg = lax.dot_general(
        degp_ref[...], ones, (((0,), (0,)), ((), ())),
        preferred_element_type=jnp.float32,
    )
    dinv_ref[...] = lax.rsqrt(deg + 1.0)


_dinv_call = pl.pallas_call(
    _dinv_body,
    out_shape=jax.ShapeDtypeStruct((N, D), jnp.float32),
)


def _mm_scale_body(x_ref, w_ref, dinv_ref, o_ref):
    o_ref[...] = (
        jnp.dot(x_ref[...], w_ref[...], preferred_element_type=jnp.float32,
                precision=lax.Precision.HIGHEST)
        * dinv_ref[...]
    )


_mm_scale = pl.pallas_call(
    _mm_scale_body,
    grid=(GRID,),
    in_specs=[
        pl.BlockSpec((RB, D), lambda i: (i, 0)),
        pl.BlockSpec((D, D), lambda i: (0, 0)),
        pl.BlockSpec((RB, D), lambda i: (i, 0)),
    ],
    out_specs=pl.BlockSpec((RB, D), lambda i: (i, 0)),
    out_shape=jax.ShapeDtypeStruct((N, D), jnp.float32),
)


def _comb_body(p_ref, hp_ref, dinv_ref, b_ref, t_ref, s1_ref, s2_ref):
    t = (p_ref[0] + p_ref[1] + hp_ref[...]) * dinv_ref[...] + b_ref[...]
    t_ref[...] = t
    s1 = jnp.sum(t, axis=0, keepdims=True)
    s2 = jnp.sum(t * t, axis=0, keepdims=True)

    @pl.when(pl.program_id(0) == 0)
    def _():
        s1_ref[...] = s1
        s2_ref[...] = s2

    @pl.when(pl.program_id(0) != 0)
    def _():
        s1_ref[...] += s1
        s2_ref[...] += s2


_comb = pl.pallas_call(
    _comb_body,
    grid=(GRID,),
    in_specs=[
        pl.BlockSpec((NC, RB, D), lambda i: (0, i, 0)),
        pl.BlockSpec((RB, D), lambda i: (i, 0)),
        pl.BlockSpec((RB, D), lambda i: (i, 0)),
        pl.BlockSpec((1, D), lambda i: (0, 0)),
    ],
    out_specs=[
        pl.BlockSpec((RB, D), lambda i: (i, 0)),
        pl.BlockSpec((1, D), lambda i: (0, 0)),
        pl.BlockSpec((1, D), lambda i: (0, 0)),
    ],
    out_shape=[
        jax.ShapeDtypeStruct((N, D), jnp.float32),
        jax.ShapeDtypeStruct((1, D), jnp.float32),
        jax.ShapeDtypeStruct((1, D), jnp.float32),
    ],
)


def _bn_prelu(t, s1, s2, g, be, a):
    mean = s1 / N
    var = s2 / N - mean * mean
    inv = lax.rsqrt(var + 1e-5)
    u = g * (t - mean) * inv + be
    return jnp.where(u >= 0, u, a * u)


def _bn_mm_body(t_ref, s1_ref, s2_ref, g_ref, be_ref, a_ref, w_ref, dinv_ref,
                o_ref):
    y = _bn_prelu(t_ref[...], s1_ref[...], s2_ref[...], g_ref[...], be_ref[...],
                  a_ref[...])
    o_ref[...] = (
        jnp.dot(y, w_ref[...], preferred_element_type=jnp.float32,
                precision=lax.Precision.HIGHEST)
        * dinv_ref[...]
    )


_bn_mm = pl.pallas_call(
    _bn_mm_body,
    grid=(GRID,),
    in_specs=[
        pl.BlockSpec((RB, D), lambda i: (i, 0)),
        pl.BlockSpec((1, D), lambda i: (0, 0)),
        pl.BlockSpec((1, D), lambda i: (0, 0)),
        pl.BlockSpec((1, D), lambda i: (0, 0)),
        pl.BlockSpec((1, D), lambda i: (0, 0)),
        pl.BlockSpec((1, D), lambda i: (0, 0)),
        pl.BlockSpec((D, D), lambda i: (0, 0)),
        pl.BlockSpec((RB, D), lambda i: (i, 0)),
    ],
    out_specs=pl.BlockSpec((RB, D), lambda i: (i, 0)),
    out_shape=jax.ShapeDtypeStruct((N, D), jnp.float32),
)


def _bn_out_body(t_ref, s1_ref, s2_ref, g_ref, be_ref, a_ref, o_ref):
    o_ref[...] = _bn_prelu(t_ref[...], s1_ref[...], s2_ref[...], g_ref[...],
                           be_ref[...], a_ref[...])


_bn_out = pl.pallas_call(
    _bn_out_body,
    grid=(GRID,),
    in_specs=[
        pl.BlockSpec((RB, D), lambda i: (i, 0)),
        pl.BlockSpec((1, D), lambda i: (0, 0)),
        pl.BlockSpec((1, D), lambda i: (0, 0)),
        pl.BlockSpec((1, D), lambda i: (0, 0)),
        pl.BlockSpec((1, D), lambda i: (0, 0)),
        pl.BlockSpec((1, D), lambda i: (0, 0)),
    ],
    out_specs=pl.BlockSpec((RB, D), lambda i: (i, 0)),
    out_shape=jax.ShapeDtypeStruct((N, D), jnp.float32),
)


def kernel(x, edge_index, W1, b1, gamma1, beta1, a1, W2, b2, gamma2, beta2, a2):
    src = edge_index[0]
    dst = edge_index[1]
    dst_deg = dst.reshape(NW, DEG_R, DEG_C)
    src_msg = jnp.concatenate(
        [src.reshape(NW, EPW), jnp.zeros((NW, PAD), jnp.int32)], axis=1
    ).reshape(NW, MSG_R, MSG_C)
    dst_msg = jnp.concatenate(
        [dst.reshape(NW, EPW), jnp.full((NW, PAD), N, jnp.int32)], axis=1
    ).reshape(NW, MSG_R, MSG_C)

    degp = _deg_kernel(dst_deg).reshape(NW, N)
    dinv = _dinv_call(degp)

    b1r = b1.reshape(1, D)
    g1r = gamma1.reshape(1, D)
    be1r = beta1.reshape(1, D)
    a1r = jnp.broadcast_to(a1.reshape(1, 1), (1, D))
    b2r = b2.reshape(1, D)
    g2r = gamma2.reshape(1, D)
    be2r = beta2.reshape(1, D)
    a2r = jnp.broadcast_to(a2.reshape(1, 1), (1, D))

    hp1 = _mm_scale(x, W1, dinv)
    p1 = _msg_kernel(hp1, src_msg, dst_msg).reshape(NC, N, D)
    t1, s1a, s2a = _comb(p1, hp1, dinv, b1r)
    hp2 = _bn_mm(t1, s1a, s2a, g1r, be1r, a1r, W2, dinv)
    p2 = _msg_kernel(hp2, src_msg, dst_msg).reshape(NC, N, D)
    t2, s1b, s2b = _comb(p2, hp2, dinv, b2r)
    return _bn_out(t2, s1b, s2b, g2r, be2r, a2r)


# speedup vs baseline: 2.0441x; 2.0441x over previous
import jax
import jax.numpy as jnp
from jax import lax
from jax.experimental import pallas as pl
from jax.experimental.pallas import tpu as pltpu
from jax.experimental.pallas import tpu_sc as plsc

N = 10000
E = 320000
D = 128

NC = 2
NS = 16
NW = NC * NS
EPW = E // NW

DEG_C = 80
DEG_R = EPW // DEG_C

MSG_C = 125
MSG_R = EPW // MSG_C
RPT = N // NS

_mesh = plsc.VectorSubcoreMesh(
    core_axis_name="c", subcore_axis_name="s", num_cores=NC, num_subcores=NS
)


def _deg_body(dst_hbm, degp_hbm, idx_v, deg_v, sem):
    cid = lax.axis_index("c")
    sid = lax.axis_index("s")
    w = cid * NS + sid
    pltpu.async_copy(dst_hbm.at[w], idx_v, sem).wait()

    zero16 = jnp.zeros((16,), jnp.float32)

    def zloop(i, carry):
        deg_v[pl.ds(i * 16, 16)] = zero16
        return carry

    lax.fori_loop(0, N // 16, zloop, 0)

    ones16 = jnp.ones((16,), jnp.float32)

    def eloop(r, carry):
        for g in range(DEG_C // 16):
            idx16 = idx_v[r, pl.ds(g * 16, 16)]
            plsc.addupdate_scatter(deg_v, [idx16], ones16)
        return carry

    lax.fori_loop(0, DEG_R, eloop, 0)
    pltpu.sync_copy(deg_v, degp_hbm.at[pl.ds(w * N, N)])


_deg_kernel = pl.kernel(
    _deg_body,
    out_type=jax.ShapeDtypeStruct((NW * N,), jnp.float32),
    mesh=_mesh,
    compiler_params=pltpu.CompilerParams(needs_layout_passes=False),
    scratch_types=[
        pltpu.VMEM((DEG_R, DEG_C), jnp.int32),
        pltpu.VMEM((N,), jnp.float32),
        pltpu.SemaphoreType.DMA,
    ],
)


ZCH = 16
RPT_A = 624
ZCH_N = RPT_A // ZCH


def _msg_body(hp_hbm, src_hbm, dst_hbm, out_hbm, sidx_v, didx_v, rows_v,
              agg_s, sem):
    cid = lax.axis_index("c")
    sid = lax.axis_index("s")
    w = cid * NS + sid
    pltpu.async_copy(src_hbm.at[w], sidx_v, sem)
    pltpu.async_copy(dst_hbm.at[w], didx_v, sem)

    zero16 = jnp.zeros((16,), jnp.float32)

    def zloop(i, carry):
        r = i // (D // 16)
        c = (i % (D // 16)) * 16
        rows_v[r, pl.ds(c, 16)] = zero16
        return carry

    lax.fori_loop(0, MSG_C * (D // 16), zloop, 0)
    base = sid * RPT_A
    nfull = RPT_A // MSG_C
    rem = RPT_A - nfull * MSG_C
    for k in range(nfull):
        pltpu.sync_copy(rows_v, agg_s.at[pl.ds(base + k * MSG_C, MSG_C)])
    pltpu.sync_copy(
        rows_v.at[pl.ds(0, rem)], agg_s.at[pl.ds(base + nfull * MSG_C, rem)]
    )

    @pl.when(sid == NS - 1)
    def _():
        pltpu.sync_copy(rows_v.at[pl.ds(0, ZCH)], agg_s.at[pl.ds(N - ZCH, ZCH)])

    pltpu.make_async_copy(src_hbm.at[w], sidx_v, sem).wait()
    pltpu.make_async_copy(dst_hbm.at[w], didx_v, sem).wait()
    plsc.subcore_barrier()

    def mloop(k, carry):
        pltpu.async_copy(hp_hbm.at[sidx_v.at[k]], rows_v, sem).wait()
        pltpu.sync_copy(rows_v, agg_s.at[didx_v.at[k]], add=True)
        return carry

    lax.fori_loop(0, MSG_R, mloop, 0)
    plsc.subcore_barrier()
    pltpu.sync_copy(
        agg_s.at[pl.ds(base, RPT_A)],
        out_hbm.at[pl.ds(cid * N + base, RPT_A)],
    )

    @pl.when(sid == NS - 1)
    def _():
        pltpu.sync_copy(
            agg_s.at[pl.ds(N - ZCH, ZCH)],
            out_hbm.at[pl.ds(cid * N + N - ZCH, ZCH)],
        )


_msg_kernel = pl.kernel(
    _msg_body,
    out_type=jax.ShapeDtypeStruct((NC * N, D), jnp.float32),
    mesh=_mesh,
    scratch_types=[
        pltpu.VMEM((MSG_R, MSG_C), jnp.int32),
        pltpu.VMEM((MSG_R, MSG_C), jnp.int32),
        pltpu.VMEM((MSG_C, D), jnp.float32),
        pltpu.VMEM_SHARED((N, D), jnp.float32),
        pltpu.SemaphoreType.DMA,
    ],
)


RB = 1000
GRID = N // RB


def _dinv_body(degp_ref, dinv_ref):
    ones = jnp.ones((NW, D), jnp.float32)
    deg = lax.dot_general(
        degp_ref[...], ones, (((0,), (0,)), ((), ())),
        preferred_element_type=jnp.float32,
    )
    dinv_ref[...] = lax.rsqrt(deg + 1.0)


_dinv_call = pl.pallas_call(
    _dinv_body,
    out_shape=jax.ShapeDtypeStruct((N, D), jnp.float32),
)


def _mm_scale_body(x_ref, w_ref, dinv_ref, o_ref):
    o_ref[...] = (
        jnp.dot(x_ref[...], w_ref[...], preferred_element_type=jnp.float32,
                precision=lax.Precision.HIGHEST)
        * dinv_ref[...]
    )


_mm_scale = pl.pallas_call(
    _mm_scale_body,
    grid=(GRID,),
    in_specs=[
        pl.BlockSpec((RB, D), lambda i: (i, 0)),
        pl.BlockSpec((D, D), lambda i: (0, 0)),
        pl.BlockSpec((RB, D), lambda i: (i, 0)),
    ],
    out_specs=pl.BlockSpec((RB, D), lambda i: (i, 0)),
    out_shape=jax.ShapeDtypeStruct((N, D), jnp.float32),
)


def _comb_body(p_ref, hp_ref, dinv_ref, b_ref, t_ref, s1_ref, s2_ref):
    t = (p_ref[0] + p_ref[1] + hp_ref[...]) * dinv_ref[...] + b_ref[...]
    t_ref[...] = t
    s1 = jnp.sum(t, axis=0, keepdims=True)
    s2 = jnp.sum(t * t, axis=0, keepdims=True)

    @pl.when(pl.program_id(0) == 0)
    def _():
        s1_ref[...] = s1
        s2_ref[...] = s2

    @pl.when(pl.program_id(0) != 0)
    def _():
        s1_ref[...] += s1
        s2_ref[...] += s2


_comb = pl.pallas_call(
    _comb_body,
    grid=(GRID,),
    in_specs=[
        pl.BlockSpec((NC, RB, D), lambda i: (0, i, 0)),
        pl.BlockSpec((RB, D), lambda i: (i, 0)),
        pl.BlockSpec((RB, D), lambda i: (i, 0)),
        pl.BlockSpec((1, D), lambda i: (0, 0)),
    ],
    out_specs=[
        pl.BlockSpec((RB, D), lambda i: (i, 0)),
        pl.BlockSpec((1, D), lambda i: (0, 0)),
        pl.BlockSpec((1, D), lambda i: (0, 0)),
    ],
    out_shape=[
        jax.ShapeDtypeStruct((N, D), jnp.float32),
        jax.ShapeDtypeStruct((1, D), jnp.float32),
        jax.ShapeDtypeStruct((1, D), jnp.float32),
    ],
)


def _bn_prelu(t, s1, s2, g, be, a):
    mean = s1 / N
    var = s2 / N - mean * mean
    inv = lax.rsqrt(var + 1e-5)
    u = g * (t - mean) * inv + be
    return jnp.where(u >= 0, u, a * u)


def _bn_mm_body(t_ref, s1_ref, s2_ref, g_ref, be_ref, a_ref, w_ref, dinv_ref,
                o_ref):
    y = _bn_prelu(t_ref[...], s1_ref[...], s2_ref[...], g_ref[...], be_ref[...],
                  a_ref[...])
    o_ref[...] = (
        jnp.dot(y, w_ref[...], preferred_element_type=jnp.float32,
                precision=lax.Precision.HIGHEST)
        * dinv_ref[...]
    )


_bn_mm = pl.pallas_call(
    _bn_mm_body,
    grid=(GRID,),
    in_specs=[
        pl.BlockSpec((RB, D), lambda i: (i, 0)),
        pl.BlockSpec((1, D), lambda i: (0, 0)),
        pl.BlockSpec((1, D), lambda i: (0, 0)),
        pl.BlockSpec((1, D), lambda i: (0, 0)),
        pl.BlockSpec((1, D), lambda i: (0, 0)),
        pl.BlockSpec((1, D), lambda i: (0, 0)),
        pl.BlockSpec((D, D), lambda i: (0, 0)),
        pl.BlockSpec((RB, D), lambda i: (i, 0)),
    ],
    out_specs=pl.BlockSpec((RB, D), lambda i: (i, 0)),
    out_shape=jax.ShapeDtypeStruct((N, D), jnp.float32),
)


def _bn_out_body(t_ref, s1_ref, s2_ref, g_ref, be_ref, a_ref, o_ref):
    o_ref[...] = _bn_prelu(t_ref[...], s1_ref[...], s2_ref[...], g_ref[...],
                           be_ref[...], a_ref[...])


_bn_out = pl.pallas_call(
    _bn_out_body,
    grid=(GRID,),
    in_specs=[
        pl.BlockSpec((RB, D), lambda i: (i, 0)),
        pl.BlockSpec((1, D), lambda i: (0, 0)),
        pl.BlockSpec((1, D), lambda i: (0, 0)),
        pl.BlockSpec((1, D), lambda i: (0, 0)),
        pl.BlockSpec((1, D), lambda i: (0, 0)),
        pl.BlockSpec((1, D), lambda i: (0, 0)),
    ],
    out_specs=pl.BlockSpec((RB, D), lambda i: (i, 0)),
    out_shape=jax.ShapeDtypeStruct((N, D), jnp.float32),
)


def kernel(x, edge_index, W1, b1, gamma1, beta1, a1, W2, b2, gamma2, beta2, a2):
    src = edge_index[0]
    dst = edge_index[1]
    dst_deg = dst.reshape(NW, DEG_R, DEG_C)
    src_msg = src.reshape(NW, MSG_R, MSG_C)
    dst_msg = dst.reshape(NW, MSG_R, MSG_C)

    degp = _deg_kernel(dst_deg).reshape(NW, N)
    dinv = _dinv_call(degp)

    b1r = b1.reshape(1, D)
    g1r = gamma1.reshape(1, D)
    be1r = beta1.reshape(1, D)
    a1r = jnp.broadcast_to(a1.reshape(1, 1), (1, D))
    b2r = b2.reshape(1, D)
    g2r = gamma2.reshape(1, D)
    be2r = beta2.reshape(1, D)
    a2r = jnp.broadcast_to(a2.reshape(1, 1), (1, D))

    hp1 = _mm_scale(x, W1, dinv)
    p1 = _msg_kernel(hp1, src_msg, dst_msg).reshape(NC, N, D)
    t1, s1a, s2a = _comb(p1, hp1, dinv, b1r)
    hp2 = _bn_mm(t1, s1a, s2a, g1r, be1r, a1r, W2, dinv)
    p2 = _msg_kernel(hp2, src_msg, dst_msg).reshape(NC, N, D)
    t2, s1b, s2b = _comb(p2, hp2, dinv, b2r)
    return _bn_out(t2, s1b, s2b, g2r, be2r, a2r)


# speedup vs baseline: 2.0459x; 1.0009x over previous
import jax
import jax.numpy as jnp
from jax import lax
from jax.experimental import pallas as pl
from jax.experimental.pallas import tpu as pltpu
from jax.experimental.pallas import tpu_sc as plsc

N = 10000
E = 320000
D = 128

NC = 2
NS = 16
NW = NC * NS
EPW = E // NW

DEG_C = 80
DEG_R = EPW // DEG_C

MSG_C = 125
MSG_R = EPW // MSG_C
RPT = N // NS

_mesh = plsc.VectorSubcoreMesh(
    core_axis_name="c", subcore_axis_name="s", num_cores=NC, num_subcores=NS
)


def _deg_body(dst_hbm, degp_hbm, idx_v, deg_v, sem):
    cid = lax.axis_index("c")
    sid = lax.axis_index("s")
    w = cid * NS + sid
    pltpu.async_copy(dst_hbm.at[w], idx_v, sem).wait()

    zero16 = jnp.zeros((16,), jnp.float32)

    def zloop(i, carry):
        deg_v[pl.ds(i * 16, 16)] = zero16
        return carry

    lax.fori_loop(0, N // 16, zloop, 0)

    ones16 = jnp.ones((16,), jnp.float32)

    def eloop(r, carry):
        for g in range(DEG_C // 16):
            idx16 = idx_v[r, pl.ds(g * 16, 16)]
            plsc.addupdate_scatter(deg_v, [idx16], ones16)
        return carry

    lax.fori_loop(0, DEG_R, eloop, 0)
    pltpu.sync_copy(deg_v, degp_hbm.at[pl.ds(w * N, N)])


_deg_kernel = pl.kernel(
    _deg_body,
    out_type=jax.ShapeDtypeStruct((NW * N,), jnp.float32),
    mesh=_mesh,
    compiler_params=pltpu.CompilerParams(needs_layout_passes=False),
    scratch_types=[
        pltpu.VMEM((DEG_R, DEG_C), jnp.int32),
        pltpu.VMEM((N,), jnp.float32),
        pltpu.SemaphoreType.DMA,
    ],
)


ZCH = 16
RPT_A = 624


def _msg_body(hp_hbm, src_hbm, dst_hbm, out_hbm, sidx_v, didx_v, rows_v,
              agg_s, sem):
    cid = lax.axis_index("c")
    sid = lax.axis_index("s")
    w = cid * NS + sid
    pltpu.async_copy(src_hbm.at[w], sidx_v, sem)
    pltpu.async_copy(dst_hbm.at[w], didx_v, sem)

    zero16 = jnp.zeros((16,), jnp.float32)

    def zloop(i, carry):
        r = i // (D // 16)
        c = (i % (D // 16)) * 16
        rows_v[r, pl.ds(c, 16)] = zero16
        return carry

    lax.fori_loop(0, MSG_C * (D // 16), zloop, 0)
    base = sid * RPT_A
    nfull = RPT_A // MSG_C
    rem = RPT_A - nfull * MSG_C
    for k in range(nfull):
        pltpu.sync_copy(rows_v, agg_s.at[pl.ds(base + k * MSG_C, MSG_C)])
    pltpu.sync_copy(
        rows_v.at[pl.ds(0, rem)], agg_s.at[pl.ds(base + nfull * MSG_C, rem)]
    )

    @pl.when(sid == NS - 1)
    def _():
        pltpu.sync_copy(rows_v.at[pl.ds(0, ZCH)], agg_s.at[pl.ds(N - ZCH, ZCH)])

    pltpu.make_async_copy(src_hbm.at[w], sidx_v, sem).wait()
    pltpu.make_async_copy(dst_hbm.at[w], didx_v, sem).wait()
    plsc.subcore_barrier()

    def mloop(k, carry):
        pltpu.async_copy(hp_hbm.at[sidx_v.at[k]], rows_v, sem).wait()
        pltpu.sync_copy(rows_v, agg_s.at[didx_v.at[k]], add=True)
        return carry

    lax.fori_loop(0, MSG_R, mloop, 0)
    plsc.subcore_barrier()
    pltpu.sync_copy(
        agg_s.at[pl.ds(base, RPT_A)],
        out_hbm.at[pl.ds(cid * N + base, RPT_A)],
    )

    @pl.when(sid == NS - 1)
    def _():
        pltpu.sync_copy(
            agg_s.at[pl.ds(N - ZCH, ZCH)],
            out_hbm.at[pl.ds(cid * N + N - ZCH, ZCH)],
        )


_msg_kernel = pl.kernel(
    _msg_body,
    out_type=jax.ShapeDtypeStruct((NC * N, D), jnp.float32),
    mesh=_mesh,
    scratch_types=[
        pltpu.VMEM((MSG_R, MSG_C), jnp.int32),
        pltpu.VMEM((MSG_R, MSG_C), jnp.int32),
        pltpu.VMEM((MSG_C, D), jnp.float32),
        pltpu.VMEM_SHARED((N, D), jnp.float32),
        pltpu.SemaphoreType.DMA,
    ],
)


RB = 1000
GRID = N // RB


def _dinv_body(degp_ref, dinv_ref):
    ones = jnp.ones((NW, D), jnp.float32)
    deg = lax.dot_general(
        degp_ref[...], ones, (((0,), (0,)), ((), ())),
        preferred_element_type=jnp.float32,
    )
    dinv_ref[...] = lax.rsqrt(deg + 1.0)


_dinv_call = pl.pallas_call(
    _dinv_body,
    out_shape=jax.ShapeDtypeStruct((N, D), jnp.float32),
)


def _mm_scale_body(x_ref, w_ref, dinv_ref, o_ref):
    o_ref[...] = (
        jnp.dot(x_ref[...], w_ref[...], preferred_element_type=jnp.float32,
                precision=lax.Precision.HIGHEST)
        * dinv_ref[...]
    )


_mm_scale = pl.pallas_call(
    _mm_scale_body,
    grid=(GRID,),
    in_specs=[
        pl.BlockSpec((RB, D), lambda i: (i, 0)),
        pl.BlockSpec((D, D), lambda i: (0, 0)),
        pl.BlockSpec((RB, D), lambda i: (i, 0)),
    ],
    out_specs=pl.BlockSpec((RB, D), lambda i: (i, 0)),
    out_shape=jax.ShapeDtypeStruct((N, D), jnp.float32),
)


def _comb_body(p_ref, hp_ref, dinv_ref, b_ref, t_ref, s1_ref, s2_ref):
    t = (p_ref[0] + p_ref[1] + hp_ref[...]) * dinv_ref[...] + b_ref[...]
    t_ref[...] = t
    s1 = jnp.sum(t, axis=0, keepdims=True)
    s2 = jnp.sum(t * t, axis=0, keepdims=True)

    @pl.when(pl.program_id(0) == 0)
    def _():
        s1_ref[...] = s1
        s2_ref[...] = s2

    @pl.when(pl.program_id(0) != 0)
    def _():
        s1_ref[...] += s1
        s2_ref[...] += s2


_comb = pl.pallas_call(
    _comb_body,
    grid=(GRID,),
    in_specs=[
        pl.BlockSpec((NC, RB, D), lambda i: (0, i, 0)),
        pl.BlockSpec((RB, D), lambda i: (i, 0)),
        pl.BlockSpec((RB, D), lambda i: (i, 0)),
        pl.BlockSpec((1, D), lambda i: (0, 0)),
    ],
    out_specs=[
        pl.BlockSpec((RB, D), lambda i: (i, 0)),
        pl.BlockSpec((1, D), lambda i: (0, 0)),
        pl.BlockSpec((1, D), lambda i: (0, 0)),
    ],
    out_shape=[
        jax.ShapeDtypeStruct((N, D), jnp.float32),
        jax.ShapeDtypeStruct((1, D), jnp.float32),
        jax.ShapeDtypeStruct((1, D), jnp.float32),
    ],
)


def _bn_prelu(t, s1, s2, g, be, a):
    mean = s1 / N
    var = s2 / N - mean * mean
    inv = lax.rsqrt(var + 1e-5)
    u = g * (t - mean) * inv + be
    return jnp.where(u >= 0, u, a * u)


def _bn_mm_body(t_ref, s1_ref, s2_ref, g_ref, be_ref, a_ref, w_ref, dinv_ref,
                o_ref):
    y = _bn_prelu(t_ref[...], s1_ref[...], s2_ref[...], g_ref[...], be_ref[...],
                  a_ref[...])
    o_ref[...] = (
        jnp.dot(y, w_ref[...], preferred_element_type=jnp.float32,
                precision=lax.Precision.HIGHEST)
        * dinv_ref[...]
    )


_bn_mm = pl.pallas_call(
    _bn_mm_body,
    grid=(GRID,),
    in_specs=[
        pl.BlockSpec((RB, D), lambda i: (i, 0)),
        pl.BlockSpec((1, D), lambda i: (0, 0)),
        pl.BlockSpec((1, D), lambda i: (0, 0)),
        pl.BlockSpec((1, D), lambda i: (0, 0)),
        pl.BlockSpec((1, D), lambda i: (0, 0)),
        pl.BlockSpec((1, D), lambda i: (0, 0)),
        pl.BlockSpec((D, D), lambda i: (0, 0)),
        pl.BlockSpec((RB, D), lambda i: (i, 0)),
    ],
    out_specs=pl.BlockSpec((RB, D), lambda i: (i, 0)),
    out_shape=jax.ShapeDtypeStruct((N, D), jnp.float32),
)


def _bn_out_body(t_ref, s1_ref, s2_ref, g_ref, be_ref, a_ref, o_ref):
    o_ref[...] = _bn_prelu(t_ref[...], s1_ref[...], s2_ref[...], g_ref[...],
                           be_ref[...], a_ref[...])


_bn_out = pl.pallas_call(
    _bn_out_body,
    grid=(GRID,),
    in_specs=[
        pl.BlockSpec((RB, D), lambda i: (i, 0)),
        pl.BlockSpec((1, D), lambda i: (0, 0)),
        pl.BlockSpec((1, D), lambda i: (0, 0)),
        pl.BlockSpec((1, D), lambda i: (0, 0)),
        pl.BlockSpec((1, D), lambda i: (0, 0)),
        pl.BlockSpec((1, D), lambda i: (0, 0)),
    ],
    out_specs=pl.BlockSpec((RB, D), lambda i: (i, 0)),
    out_shape=jax.ShapeDtypeStruct((N, D), jnp.float32),
)


def kernel(x, edge_index, W1, b1, gamma1, beta1, a1, W2, b2, gamma2, beta2, a2):
    src = edge_index[0]
    dst = edge_index[1]
    dst_deg = dst.reshape(NW, DEG_R, DEG_C)
    src_msg = src.reshape(NW, MSG_R, MSG_C)
    dst_msg = dst.reshape(NW, MSG_R, MSG_C)

    degp = _deg_kernel(dst_deg).reshape(NW, N)
    dinv = _dinv_call(degp)

    b1r = b1.reshape(1, D)
    g1r = gamma1.reshape(1, D)
    be1r = beta1.reshape(1, D)
    a1r = jnp.broadcast_to(a1.reshape(1, 1), (1, D))
    b2r = b2.reshape(1, D)
    g2r = gamma2.reshape(1, D)
    be2r = beta2.reshape(1, D)
    a2r = jnp.broadcast_to(a2.reshape(1, 1), (1, D))

    hp1 = _mm_scale(x, W1, dinv)
    p1 = _msg_kernel(hp1, src_msg, dst_msg).reshape(NC, N, D)
    t1, s1a, s2a = _comb(p1, hp1, dinv, b1r)
    hp2 = _bn_mm(t1, s1a, s2a, g1r, be1r, a1r, W2, dinv)
    p2 = _msg_kernel(hp2, src_msg, dst_msg).reshape(NC, N, D)
    t2, s1b, s2b = _comb(p2, hp2, dinv, b2r)
    return _bn_out(t2, s1b, s2b, g2r, be2r, a2r)


# speedup vs baseline: 2.0615x; 1.0076x over previous
import jax
import jax.numpy as jnp
from jax import lax
from jax.experimental import pallas as pl
from jax.experimental.pallas import tpu as pltpu
from jax.experimental.pallas import tpu_sc as plsc

N = 10000
E = 320000
D = 128

NC = 2
NS = 16
NW = NC * NS
EPW = E // NW

DEG_C = 80
DEG_R = EPW // DEG_C

MSG_C = 125
MSG_R = EPW // MSG_C
RPT = N // NS

_mesh = plsc.VectorSubcoreMesh(
    core_axis_name="c", subcore_axis_name="s", num_cores=NC, num_subcores=NS
)


def _deg_body(dst_hbm, degp_hbm, idx_v, deg_v, sem):
    cid = lax.axis_index("c")
    sid = lax.axis_index("s")
    w = cid * NS + sid
    pltpu.async_copy(dst_hbm.at[w], idx_v, sem).wait()

    zero16 = jnp.zeros((16,), jnp.float32)

    def zloop(i, carry):
        deg_v[pl.ds(i * 16, 16)] = zero16
        return carry

    lax.fori_loop(0, N // 16, zloop, 0)

    ones16 = jnp.ones((16,), jnp.float32)

    def eloop(r, carry):
        for g in range(DEG_C // 16):
            idx16 = idx_v[r, pl.ds(g * 16, 16)]
            plsc.addupdate_scatter(deg_v, [idx16], ones16)
        return carry

    lax.fori_loop(0, DEG_R, eloop, 0)
    pltpu.sync_copy(deg_v, degp_hbm.at[pl.ds(w * N, N)])


_deg_kernel = pl.kernel(
    _deg_body,
    out_type=jax.ShapeDtypeStruct((NW * N,), jnp.float32),
    mesh=_mesh,
    compiler_params=pltpu.CompilerParams(needs_layout_passes=False),
    scratch_types=[
        pltpu.VMEM((DEG_R, DEG_C), jnp.int32),
        pltpu.VMEM((N,), jnp.float32),
        pltpu.SemaphoreType.DMA,
    ],
)


ZCH = 16
RPT_A = 624


def _msg_body(hp_hbm, src_hbm, dst_hbm, out_hbm, sidx_v, didx_v, rows_v,
              agg_s, sem):
    cid = lax.axis_index("c")
    sid = lax.axis_index("s")
    w = cid * NS + sid
    pltpu.async_copy(src_hbm.at[w], sidx_v, sem)
    pltpu.async_copy(dst_hbm.at[w], didx_v, sem)

    zero16 = jnp.zeros((16,), jnp.float32)

    def zloop(i, carry):
        r = i // (D // 16)
        c = (i % (D // 16)) * 16
        rows_v[r, pl.ds(c, 16)] = zero16
        return carry

    lax.fori_loop(0, MSG_C * (D // 16), zloop, 0)
    base = sid * RPT_A
    nfull = RPT_A // MSG_C
    rem = RPT_A - nfull * MSG_C
    for k in range(nfull):
        pltpu.sync_copy(rows_v, agg_s.at[pl.ds(base + k * MSG_C, MSG_C)])
    pltpu.sync_copy(
        rows_v.at[pl.ds(0, rem)], agg_s.at[pl.ds(base + nfull * MSG_C, rem)]
    )

    @pl.when(sid == NS - 1)
    def _():
        pltpu.sync_copy(rows_v.at[pl.ds(0, ZCH)], agg_s.at[pl.ds(N - ZCH, ZCH)])

    pltpu.make_async_copy(src_hbm.at[w], sidx_v, sem).wait()
    pltpu.make_async_copy(dst_hbm.at[w], didx_v, sem).wait()
    plsc.subcore_barrier()

    def mloop(k, carry):
        pltpu.async_copy(hp_hbm.at[sidx_v.at[k]], rows_v, sem).wait()
        pltpu.sync_copy(rows_v, agg_s.at[didx_v.at[k]], add=True)
        return carry

    lax.fori_loop(0, MSG_R, mloop, 0)
    plsc.subcore_barrier()
    pltpu.sync_copy(
        agg_s.at[pl.ds(base, RPT_A)],
        out_hbm.at[pl.ds(cid * N + base, RPT_A)],
    )

    @pl.when(sid == NS - 1)
    def _():
        pltpu.sync_copy(
            agg_s.at[pl.ds(N - ZCH, ZCH)],
            out_hbm.at[pl.ds(cid * N + N - ZCH, ZCH)],
        )


_msg_kernel = pl.kernel(
    _msg_body,
    out_type=jax.ShapeDtypeStruct((NC * N, D), jnp.float32),
    mesh=_mesh,
    scratch_types=[
        pltpu.VMEM((MSG_R, MSG_C), jnp.int32),
        pltpu.VMEM((MSG_R, MSG_C), jnp.int32),
        pltpu.VMEM((MSG_C, D), jnp.float32),
        pltpu.VMEM_SHARED((N, D), jnp.float32),
        pltpu.SemaphoreType.DMA,
    ],
)


RB = 1000
GRID = N // RB


def _mm_scale_body(degp_ref, x_ref, w_ref, hp_ref, dinv_ref, dfull_ref):
    @pl.when(pl.program_id(0) == 0)
    def _():
        ones = jnp.ones((NW, D), jnp.float32)
        deg = lax.dot_general(
            degp_ref[...], ones, (((0,), (0,)), ((), ())),
            preferred_element_type=jnp.float32,
        )
        dfull_ref[...] = lax.rsqrt(deg + 1.0)

    dinv = dfull_ref[pl.ds(pl.program_id(0) * RB, RB), :]
    dinv_ref[...] = dinv
    hp_ref[...] = (
        jnp.dot(x_ref[...], w_ref[...], preferred_element_type=jnp.float32,
                precision=lax.Precision.HIGHEST)
        * dinv
    )


_mm_scale = pl.pallas_call(
    _mm_scale_body,
    grid=(GRID,),
    in_specs=[
        pl.BlockSpec((NW, N), lambda i: (0, 0)),
        pl.BlockSpec((RB, D), lambda i: (i, 0)),
        pl.BlockSpec((D, D), lambda i: (0, 0)),
    ],
    out_specs=[
        pl.BlockSpec((RB, D), lambda i: (i, 0)),
        pl.BlockSpec((RB, D), lambda i: (i, 0)),
    ],
    out_shape=[
        jax.ShapeDtypeStruct((N, D), jnp.float32),
        jax.ShapeDtypeStruct((N, D), jnp.float32),
    ],
    scratch_shapes=[pltpu.VMEM((N, D), jnp.float32)],
)


def _comb_body(p_ref, hp_ref, dinv_ref, b_ref, t_ref, s1_ref, s2_ref):
    t = (p_ref[0] + p_ref[1] + hp_ref[...]) * dinv_ref[...] + b_ref[...]
    t_ref[...] = t
    s1 = jnp.sum(t, axis=0, keepdims=True)
    s2 = jnp.sum(t * t, axis=0, keepdims=True)

    @pl.when(pl.program_id(0) == 0)
    def _():
        s1_ref[...] = s1
        s2_ref[...] = s2

    @pl.when(pl.program_id(0) != 0)
    def _():
        s1_ref[...] += s1
        s2_ref[...] += s2


_comb = pl.pallas_call(
    _comb_body,
    grid=(GRID,),
    in_specs=[
        pl.BlockSpec((NC, RB, D), lambda i: (0, i, 0)),
        pl.BlockSpec((RB, D), lambda i: (i, 0)),
        pl.BlockSpec((RB, D), lambda i: (i, 0)),
        pl.BlockSpec((1, D), lambda i: (0, 0)),
    ],
    out_specs=[
        pl.BlockSpec((RB, D), lambda i: (i, 0)),
        pl.BlockSpec((1, D), lambda i: (0, 0)),
        pl.BlockSpec((1, D), lambda i: (0, 0)),
    ],
    out_shape=[
        jax.ShapeDtypeStruct((N, D), jnp.float32),
        jax.ShapeDtypeStruct((1, D), jnp.float32),
        jax.ShapeDtypeStruct((1, D), jnp.float32),
    ],
)


def _bn_prelu(t, s1, s2, g, be, a):
    mean = s1 / N
    var = s2 / N - mean * mean
    inv = lax.rsqrt(var + 1e-5)
    u = g * (t - mean) * inv + be
    return jnp.where(u >= 0, u, a * u)


def _bn_mm_body(t_ref, s1_ref, s2_ref, g_ref, be_ref, a_ref, w_ref, dinv_ref,
                o_ref):
    y = _bn_prelu(t_ref[...], s1_ref[...], s2_ref[...], g_ref[...], be_ref[...],
                  a_ref[...])
    o_ref[...] = (
        jnp.dot(y, w_ref[...], preferred_element_type=jnp.float32,
                precision=lax.Precision.HIGHEST)
        * dinv_ref[...]
    )


_bn_mm = pl.pallas_call(
    _bn_mm_body,
    grid=(GRID,),
    in_specs=[
        pl.BlockSpec((RB, D), lambda i: (i, 0)),
        pl.BlockSpec((1, D), lambda i: (0, 0)),
        pl.BlockSpec((1, D), lambda i: (0, 0)),
        pl.BlockSpec((1, D), lambda i: (0, 0)),
        pl.BlockSpec((1, D), lambda i: (0, 0)),
        pl.BlockSpec((1, D), lambda i: (0, 0)),
        pl.BlockSpec((D, D), lambda i: (0, 0)),
        pl.BlockSpec((RB, D), lambda i: (i, 0)),
    ],
    out_specs=pl.BlockSpec((RB, D), lambda i: (i, 0)),
    out_shape=jax.ShapeDtypeStruct((N, D), jnp.float32),
)


def _bn_out_body(t_ref, s1_ref, s2_ref, g_ref, be_ref, a_ref, o_ref):
    o_ref[...] = _bn_prelu(t_ref[...], s1_ref[...], s2_ref[...], g_ref[...],
                           be_ref[...], a_ref[...])


_bn_out = pl.pallas_call(
    _bn_out_body,
    grid=(GRID,),
    in_specs=[
        pl.BlockSpec((RB, D), lambda i: (i, 0)),
        pl.BlockSpec((1, D), lambda i: (0, 0)),
        pl.BlockSpec((1, D), lambda i: (0, 0)),
        pl.BlockSpec((1, D), lambda i: (0, 0)),
        pl.BlockSpec((1, D), lambda i: (0, 0)),
        pl.BlockSpec((1, D), lambda i: (0, 0)),
    ],
    out_specs=pl.BlockSpec((RB, D), lambda i: (i, 0)),
    out_shape=jax.ShapeDtypeStruct((N, D), jnp.float32),
)


def kernel(x, edge_index, W1, b1, gamma1, beta1, a1, W2, b2, gamma2, beta2, a2):
    src = edge_index[0]
    dst = edge_index[1]
    dst_deg = dst.reshape(NW, DEG_R, DEG_C)
    src_msg = src.reshape(NW, MSG_R, MSG_C)
    dst_msg = dst.reshape(NW, MSG_R, MSG_C)

    degp = _deg_kernel(dst_deg).reshape(NW, N)

    b1r = b1.reshape(1, D)
    g1r = gamma1.reshape(1, D)
    be1r = beta1.reshape(1, D)
    a1r = jnp.broadcast_to(a1.reshape(1, 1), (1, D))
    b2r = b2.reshape(1, D)
    g2r = gamma2.reshape(1, D)
    be2r = beta2.reshape(1, D)
    a2r = jnp.broadcast_to(a2.reshape(1, 1), (1, D))

    hp1, dinv = _mm_scale(degp, x, W1)
    p1 = _msg_kernel(hp1, src_msg, dst_msg).reshape(NC, N, D)
    t1, s1a, s2a = _comb(p1, hp1, dinv, b1r)
    hp2 = _bn_mm(t1, s1a, s2a, g1r, be1r, a1r, W2, dinv)
    p2 = _msg_kernel(hp2, src_msg, dst_msg).reshape(NC, N, D)
    t2, s1b, s2b = _comb(p2, hp2, dinv, b2r)
    return _bn_out(t2, s1b, s2b, g2r, be2r, a2r)
